# baseline (speedup 1.0000x reference)
import functools

import jax
import jax.numpy as jnp
from jax import lax
from jax.experimental import pallas as pl
from jax.experimental.pallas import tpu as pltpu
from jax.experimental.pallas import tpu_sc as plsc

GRID_N = 1000
CELLS = GRID_N * GRID_N
EMB_D = 16
BATCH_N = 16384

NUM_CORES = 2
NUM_SUBCORES = 16
LANES = 16
SEG = 1024
NUM_SEGS = BATCH_N // SEG
CHUNK = 4096
MAIN_CHUNKS = 15
TILE_CELLS = MAIN_CHUNKS * CHUNK
EXTRA_BASE = NUM_SUBCORES * TILE_CELLS
EXTRA_LEN = 1024
TAIL_BASE = EXTRA_BASE + NUM_SUBCORES * EXTRA_LEN
TAIL_LEN = 640
SEL_CAP = BATCH_N + 32
DUMMY_COL = BATCH_N
SENTINEL = 0x7FFFFFFF
PID_BITS = 14
PID_MASK = (1 << PID_BITS) - 1

_mesh = plsc.VectorSubcoreMesh(core_axis_name="c", subcore_axis_name="s")


@functools.partial(
    pl.kernel,
    out_type=jax.ShapeDtypeStruct((2, 8, BATCH_N), jnp.float32),
    mesh=_mesh,
    compiler_params=pltpu.CompilerParams(
        use_tc_tiling_on_sc=True, needs_layout_passes=False
    ),
    scratch_types=[
        pltpu.VMEM((2, SEG), jnp.float32),
        pltpu.VMEM((SEL_CAP,), jnp.int32),
        pltpu.VMEM((SEL_CAP,), jnp.int32),
        pltpu.VMEM((2, 8, CHUNK), jnp.float32),
        pltpu.VMEM_SHARED((8, BATCH_N + 128), jnp.float32),
        pltpu.SemaphoreType.DMA,
        pltpu.SemaphoreType.DMA,
        pltpu.SemaphoreType.DMA,
    ],
)
def _grid_lookup(xt_hbm, t3_hbm, out_hbm, xs_v, pk1_v, pk2_v,
                 ring_v, val_s, sem_f0, sem_f1, sem_e):
    cid = lax.axis_index("c")
    sid = lax.axis_index("s")
    lo = sid * TILE_CELLS
    xlo = EXTRA_BASE + sid * EXTRA_LEN
    last = sid == NUM_SUBCORES - 1
    lane_iota = lax.iota(jnp.int32, LANES)
    sems = (sem_f0, sem_f1)

    n_mine = jnp.int32(0)
    for seg in range(NUM_SEGS):
        pltpu.sync_copy(xt_hbm.at[0, pl.ds(seg * SEG, SEG)], xs_v.at[0])
        pltpu.sync_copy(xt_hbm.at[1, pl.ds(seg * SEG, SEG)], xs_v.at[1])

        def scan_body(j, off, seg=seg):
            x0 = xs_v[0, pl.ds(j * LANES, LANES)]
            x1 = xs_v[1, pl.ds(j * LANES, LANES)]
            i0 = (x0 * float(GRID_N)).astype(jnp.int32)
            i1 = (x1 * float(GRID_N)).astype(jnp.int32)
            cells = i0 * GRID_N + i1
            pids = lane_iota + (seg * SEG) + j * LANES
            m_main = (cells >= lo) & (cells < lo + TILE_CELLS)
            m_extra = (cells >= xlo) & (cells < xlo + EXTRA_LEN)
            m_tail = last & (cells >= TAIL_BASE)
            local = jnp.where(m_main, cells - lo, 0)
            local = jnp.where(m_extra, cells - xlo + TILE_CELLS, local)
            local = jnp.where(
                m_tail, cells - TAIL_BASE + TILE_CELLS + EXTRA_LEN, local
            )
            m = m_main | m_extra | m_tail
            packed = (local << PID_BITS) | pids
            packed = jnp.where(m, packed, SENTINEL)
            pk1_v[pl.ds(off, LANES)] = jnp.sort(packed)
            return off + plsc.all_reduce_population_count(m)[0]

        n_mine = lax.fori_loop(0, SEG // LANES, scan_body, n_mine)

    nv_mine = (n_mine + LANES - 1) // LANES

    def issue_fetch(base, length, parity):
        off = pl.multiple_of(base, 128)
        pltpu.async_copy(
            t3_hbm.at[cid, :, pl.ds(off, length)],
            ring_v.at[parity, :, pl.ds(0, length)],
            sems[parity],
        )

    def drain_fetch(length, parity):
        pltpu.make_async_copy(
            t3_hbm.at[0, :, pl.ds(0, length)],
            ring_v.at[parity, :, pl.ds(0, length)],
            sems[parity],
        ).wait()

    def process_chunk(bc, length, parity):
        def sel_body(i, off2):
            packed = pk1_v[pl.ds(i * LANES, LANES)]
            cells = packed >> PID_BITS
            m = (
                (cells >= bc)
                & (cells < bc + length)
                & (i * LANES + lane_iota < n_mine)
            )
            packed = jnp.where(m, packed, SENTINEL)
            pk2_v[pl.ds(off2, LANES)] = jnp.sort(packed)
            return off2 + plsc.all_reduce_population_count(m)[0]

        cnt = lax.fori_loop(0, nv_mine, sel_body, jnp.int32(0))

        def ext_body(gi, carry):
            packed = pk2_v[pl.ds(gi * LANES, LANES)]
            cols = (packed >> PID_BITS) - bc
            pids = packed & PID_MASK
            valid = gi * LANES + lane_iota < cnt
            cols = jnp.where(valid, cols, 0)
            pids = jnp.where(valid, pids, DUMMY_COL)
            for k in range(LANES):
                pltpu.async_copy(
                    ring_v.at[parity, :, pl.ds(cols[k], 1)],
                    val_s.at[:, pl.ds(pids[k], 1)],
                    sem_e,
                )
            pltpu.make_async_copy(
                t3_hbm.at[0, :, pl.ds(0, LANES)],
                val_s.at[:, pl.ds(0, LANES)],
                sem_e,
            ).wait()
            return carry

        lax.fori_loop(0, (cnt + LANES - 1) // LANES, ext_body, 0)

    def issue_main(c, parity):
        issue_fetch(lo + c * CHUNK, CHUNK, parity)

    issue_main(0, 0)
    issue_main(1, 1)

    def pair_body(i, carry):
        c0 = 2 * i
        drain_fetch(CHUNK, 0)
        process_chunk(c0 * CHUNK, CHUNK, 0)
        issue_main(c0 + 2, 0)
        drain_fetch(CHUNK, 1)
        process_chunk((c0 + 1) * CHUNK, CHUNK, 1)
        issue_main(c0 + 3, 1)
        return carry

    lax.fori_loop(0, 6, pair_body, 0)
    drain_fetch(CHUNK, 0)
    process_chunk(12 * CHUNK, CHUNK, 0)
    issue_main(14, 0)
    drain_fetch(CHUNK, 1)
    process_chunk(13 * CHUNK, CHUNK, 1)
    issue_fetch(xlo, EXTRA_LEN, 1)
    drain_fetch(CHUNK, 0)
    process_chunk(14 * CHUNK, CHUNK, 0)

    @pl.when(last)
    def _():
        issue_fetch(sid * 0 + TAIL_BASE, TAIL_LEN, 0)

    drain_fetch(EXTRA_LEN, 1)
    process_chunk(TILE_CELLS, EXTRA_LEN, 1)

    @pl.when(last)
    def _():
        drain_fetch(TAIL_LEN, 0)
        process_chunk(TILE_CELLS + EXTRA_LEN, TAIL_LEN, 0)

    plsc.subcore_barrier()

    @pl.when(sid == 0)
    def _():
        pltpu.sync_copy(val_s.at[:, pl.ds(0, BATCH_N)], out_hbm.at[cid])


def kernel(x, table):
    t3 = table.T.reshape(2, 8, CELLS)
    out3 = _grid_lookup(x.T, t3)
    return out3.reshape(EMB_D, BATCH_N).T


# speedup vs baseline: 1.3712x; 1.3712x over previous
import functools

import jax
import jax.numpy as jnp
from jax import lax
from jax.experimental import pallas as pl
from jax.experimental.pallas import tpu as pltpu
from jax.experimental.pallas import tpu_sc as plsc

GRID_N = 1000
EMB_D = 16
BATCH_N = 16384

NUM_CORES = 2
NUM_SUBCORES = 16
LANES = 16
NUM_WORKERS = NUM_CORES * NUM_SUBCORES
PTS_PER_WORKER = BATCH_N // NUM_WORKERS
PTS_PER_CORE = BATCH_N // NUM_CORES
NUM_WAVES = PTS_PER_WORKER // LANES
RING = 2

_mesh = plsc.VectorSubcoreMesh(core_axis_name="c", subcore_axis_name="s")


@functools.partial(
    pl.kernel,
    out_type=jax.ShapeDtypeStruct((2, 8, BATCH_N), jnp.float32),
    mesh=_mesh,
    compiler_params=pltpu.CompilerParams(use_tc_tiling_on_sc=True),
    scratch_types=[
        pltpu.VMEM((PTS_PER_WORKER,), jnp.float32),
        pltpu.VMEM((PTS_PER_WORKER,), jnp.float32),
        pltpu.VMEM((RING * LANES, 2, 8, 128), jnp.float32),
        pltpu.VMEM((RING * LANES,), jnp.int32),
        pltpu.VMEM_SHARED((2, 8, PTS_PER_CORE), jnp.float32),
        pltpu.SemaphoreType.DMA,
        pltpu.SemaphoreType.DMA,
        pltpu.SemaphoreType.DMA,
    ],
)
def _grid_lookup(xt_hbm, t3_hbm, out_hbm, x0_v, x1_v, blk_v, lane_v, val_s,
                 sem_h0, sem_h1, sem_l):
    cid = lax.axis_index("c")
    sid = lax.axis_index("s")
    base = (cid * NUM_SUBCORES + sid) * PTS_PER_WORKER
    local = sid * PTS_PER_WORKER

    pltpu.sync_copy(xt_hbm.at[0, pl.ds(base, PTS_PER_WORKER)], x0_v)
    pltpu.sync_copy(xt_hbm.at[1, pl.ds(base, PTS_PER_WORKER)], x1_v)

    def issue_hbm(j, parity, sem):
        x0 = x0_v[pl.ds(j * LANES, LANES)]
        x1 = x1_v[pl.ds(j * LANES, LANES)]
        i0 = (x0 * float(GRID_N)).astype(jnp.int32)
        i1 = (x1 * float(GRID_N)).astype(jnp.int32)
        cells = i0 * GRID_N + i1
        slot = parity * LANES
        lane_v[pl.ds(slot, LANES)] = cells & 127
        c128 = cells & -128
        for k in range(LANES):
            off = pl.multiple_of(c128[k], 128)
            for g in range(2):
                pltpu.async_copy(
                    t3_hbm.at[g, :, pl.ds(off, 128)],
                    blk_v.at[slot + k, g],
                    sem,
                )

    def drain_extract(j, parity, sem):
        slot = parity * LANES
        for k in range(LANES):
            for g in range(2):
                pltpu.make_async_copy(
                    t3_hbm.at[g, :, pl.ds(0, 128)],
                    blk_v.at[slot + k, g],
                    sem,
                ).wait()
        lanes = lane_v[pl.ds(slot, LANES)]
        for k in range(LANES):
            pltpu.async_copy(
                blk_v.at[slot + k, :, :, pl.ds(lanes[k], 1)],
                val_s.at[:, :, pl.ds(local + j * LANES + k, 1)],
                sem_l,
            )
        pltpu.make_async_copy(
            t3_hbm.at[:, :, pl.ds(0, LANES)],
            val_s.at[:, :, pl.ds(local + j * LANES, LANES)],
            sem_l,
        ).wait()

    issue_hbm(0, 0, sem_h0)
    issue_hbm(1, 1, sem_h1)

    def body(i, carry):
        drain_extract(2 * i - 2, 0, sem_h0)
        issue_hbm(2 * i, 0, sem_h0)
        drain_extract(2 * i - 1, 1, sem_h1)
        issue_hbm(2 * i + 1, 1, sem_h1)
        return carry

    lax.fori_loop(1, NUM_WAVES // 2, body, 0)
    drain_extract(NUM_WAVES - 2, 0, sem_h0)
    drain_extract(NUM_WAVES - 1, 1, sem_h1)

    plsc.subcore_barrier()

    @pl.when(sid == 0)
    def _():
        pltpu.sync_copy(
            val_s, out_hbm.at[:, :, pl.ds(cid * PTS_PER_CORE, PTS_PER_CORE)]
        )


def kernel(x, table):
    t3 = table.T.reshape(2, 8, GRID_N * GRID_N)
    out3 = _grid_lookup(x.T, t3)
    return out3.reshape(EMB_D, BATCH_N).T
